# baseline (speedup 1.0000x reference)
import functools

import jax
import jax.numpy as jnp
from jax import lax
from jax.experimental import pallas as pl
from jax.experimental.pallas import tpu as pltpu
from jax.experimental.pallas import tpu_sc as plsc

WIDTH = 512
STEP = 256
LANES = 16


SC_G = 8
SC_NBUF = 4
SC_LOOK = 2
SC_OWAIT = 2


def _sc_body(x_hbm, out_hbm, *refs, n, ng):
    wins = refs[0:SC_NBUF]
    outs = refs[SC_NBUF:2 * SC_NBUF]
    carry_v = refs[2 * SC_NBUF]
    isems = refs[2 * SC_NBUF + 1:3 * SC_NBUF + 1]
    osems = refs[3 * SC_NBUF + 1:4 * SC_NBUF + 1]
    ch = lax.axis_index("s") * 2 + lax.axis_index("c")

    def in_cp(g, b):
        return pltpu.make_async_copy(
            x_hbm.at[pl.ds(g * SC_G, SC_G), pl.ds(ch, 1), :], wins[b], isems[b])

    def out_cp(g, b):
        return pltpu.make_async_copy(
            outs[b], out_hbm.at[pl.ds(ch, 1), pl.ds(g * SC_G * STEP, SC_G * STEP)],
            osems[b])

    zero = jnp.zeros((LANES,), jnp.float32)
    for i in range(STEP // LANES):
        carry_v[0, pl.ds(i * LANES, LANES)] = zero

    for d in range(SC_LOOK):
        in_cp(d, d).start()

    def outer(gq, _):
        for b in range(SC_NBUF):
            g = gq * SC_NBUF + b

            @pl.when(g + SC_LOOK < ng)
            def _():
                in_cp(g + SC_LOOK, (b + SC_LOOK) % SC_NBUF).start()

            in_cp(g, b).wait()

            @pl.when(g >= SC_OWAIT)
            def _():
                out_cp(g - SC_OWAIT, (b - SC_OWAIT) % SC_NBUF).wait()

            win_v = wins[b]
            out_v = outs[b]
            for k in range(SC_G):
                for i in range(STEP // LANES):
                    off = i * LANES
                    a = win_v[k, 0, pl.ds(off, LANES)]
                    if k == 0:
                        b_half = carry_v[0, pl.ds(off, LANES)]
                    else:
                        b_half = win_v[k - 1, 0, pl.ds(STEP + off, LANES)]
                    out_v[0, pl.ds(k * STEP + off, LANES)] = a + b_half
            for i in range(STEP // LANES):
                off = i * LANES
                carry_v[0, pl.ds(off, LANES)] = (
                    win_v[SC_G - 1, 0, pl.ds(STEP + off, LANES)])
            out_cp(g, b).start()
        return 0

    lax.fori_loop(0, ng // SC_NBUF, outer, 0)
    for d in range(SC_OWAIT):
        g = ng - SC_OWAIT + d
        out_cp(g, g % SC_NBUF).wait()
    pltpu.sync_copy(carry_v, out_hbm.at[pl.ds(ch, 1), pl.ds(n * STEP, STEP)])


def _sc_kernel(x):
    n, c, w = x.shape
    total = (n - 1) * STEP + w
    ng = n // SC_G
    mesh = plsc.VectorSubcoreMesh(core_axis_name="c", subcore_axis_name="s")
    kfn = pl.kernel(
        functools.partial(_sc_body, n=n, ng=ng),
        out_type=jax.ShapeDtypeStruct((c, total), x.dtype),
        mesh=mesh,
        scratch_types=(
            [pltpu.VMEM((SC_G, 1, WIDTH), jnp.float32)] * SC_NBUF
            + [pltpu.VMEM((1, SC_G * STEP), jnp.float32)] * SC_NBUF
            + [pltpu.VMEM((1, STEP), jnp.float32)]
            + [pltpu.SemaphoreType.DMA] * (2 * SC_NBUF)
        ),
    )
    return kfn(x)


SC2_G = 4
SC2_NT = 2
SC2_CPW = 2


def _sc2_body(x_hbm, out_hbm, *refs, n, ng, nw):
    wins = refs[0:SC_NBUF]
    outs = refs[SC_NBUF:2 * SC_NBUF]
    carry_v = refs[2 * SC_NBUF]
    tail_v = refs[2 * SC_NBUF + 1]
    isems = refs[2 * SC_NBUF + 2:3 * SC_NBUF + 2]
    osems = refs[3 * SC_NBUF + 2:4 * SC_NBUF + 2]
    cg = lax.axis_index("s")
    t = lax.axis_index("c")
    w0 = t * nw
    c0 = cg * SC2_CPW

    def in_cp(g, b):
        return pltpu.make_async_copy(
            x_hbm.at[pl.ds(w0 + g * SC2_G, SC2_G), pl.ds(c0, SC2_CPW), :],
            wins[b], isems[b])

    def out_cp(g, b):
        return pltpu.make_async_copy(
            outs[b],
            out_hbm.at[pl.ds(c0, SC2_CPW),
                       pl.ds((w0 + g * SC2_G) * STEP, SC2_G * STEP)],
            osems[b])

    zero = jnp.zeros((LANES,), jnp.float32)

    @pl.when(t == 0)
    def _():
        for cl in range(SC2_CPW):
            for i in range(STEP // LANES):
                carry_v[0, cl, pl.ds(i * LANES, LANES)] = zero

    @pl.when(t > 0)
    def _():
        pltpu.sync_copy(
            x_hbm.at[pl.ds(w0 - 1, 1), pl.ds(c0, SC2_CPW), pl.ds(STEP, STEP)],
            carry_v)

    for d in range(SC_LOOK):
        in_cp(d, d).start()

    def outer(gq, _):
        for b in range(SC_NBUF):
            g = gq * SC_NBUF + b

            @pl.when(g + SC_LOOK < ng)
            def _():
                in_cp(g + SC_LOOK, (b + SC_LOOK) % SC_NBUF).start()

            in_cp(g, b).wait()

            @pl.when(g >= SC_OWAIT)
            def _():
                out_cp(g - SC_OWAIT, (b - SC_OWAIT) % SC_NBUF).wait()

            win_v = wins[b]
            out_v = outs[b]
            for k in range(SC2_G):
                for cl in range(SC2_CPW):
                    for i in range(STEP // LANES):
                        off = i * LANES
                        a = win_v[k, cl, pl.ds(off, LANES)]
                        if k == 0:
                            b_half = carry_v[0, cl, pl.ds(off, LANES)]
                        else:
                            b_half = win_v[k - 1, cl, pl.ds(STEP + off, LANES)]
                        out_v[cl, pl.ds(k * STEP + off, LANES)] = a + b_half
            for cl in range(SC2_CPW):
                for i in range(STEP // LANES):
                    off = i * LANES
                    carry_v[0, cl, pl.ds(off, LANES)] = (
                        win_v[SC2_G - 1, cl, pl.ds(STEP + off, LANES)])
            out_cp(g, b).start()
        return 0

    lax.fori_loop(0, ng // SC_NBUF, outer, 0)
    for d in range(SC_OWAIT):
        g = ng - SC_OWAIT + d
        out_cp(g, g % SC_NBUF).wait()

    @pl.when(t == SC2_NT - 1)
    def _():
        for cl in range(SC2_CPW):
            for i in range(STEP // LANES):
                off = i * LANES
                tail_v[cl, pl.ds(off, LANES)] = carry_v[0, cl, pl.ds(off, LANES)]
        pltpu.sync_copy(
            tail_v, out_hbm.at[pl.ds(c0, SC2_CPW), pl.ds(n * STEP, STEP)])


def _sc2_kernel(x):
    n, c, w = x.shape
    total = (n - 1) * STEP + w
    nw = n // SC2_NT
    ng = nw // SC2_G
    mesh = plsc.VectorSubcoreMesh(core_axis_name="c", subcore_axis_name="s")
    kfn = pl.kernel(
        functools.partial(_sc2_body, n=n, ng=ng, nw=nw),
        out_type=jax.ShapeDtypeStruct((c, total), x.dtype),
        mesh=mesh,
        scratch_types=(
            [pltpu.VMEM((SC2_G, SC2_CPW, WIDTH), jnp.float32)] * SC_NBUF
            + [pltpu.VMEM((SC2_CPW, SC2_G * STEP), jnp.float32)] * SC_NBUF
            + [pltpu.VMEM((1, SC2_CPW, STEP), jnp.float32)]
            + [pltpu.VMEM((SC2_CPW, STEP), jnp.float32)]
            + [pltpu.SemaphoreType.DMA] * (2 * SC_NBUF)
        ),
    )
    return kfn(x)


G = 8


def _tc_body(x_ref, o_ref, carry_ref, *, nb):
    j = pl.program_id(0)

    @pl.when(j < nb)
    def _main():
        a0 = x_ref[0, :, :STEP]
        o_ref[:, :STEP] = jnp.where(j == 0, a0, a0 + carry_ref[...])
        for k in range(1, G):
            o_ref[:, k * STEP:(k + 1) * STEP] = (
                x_ref[k, :, :STEP] + x_ref[k - 1, :, STEP:])
        carry_ref[...] = x_ref[G - 1, :, STEP:]

    @pl.when(j == nb)
    def _tail():
        o_ref[:, :STEP] = carry_ref[...]


def _tc_kernel(x):
    n, c, w = x.shape
    total = (n - 1) * STEP + w
    nb = n // G
    return pl.pallas_call(
        functools.partial(_tc_body, nb=nb),
        grid=(nb + 1,),
        in_specs=[pl.BlockSpec((G, c, w), lambda j: (jnp.minimum(j, nb - 1), 0, 0))],
        out_specs=pl.BlockSpec((c, G * STEP), lambda j: (0, j)),
        out_shape=jax.ShapeDtypeStruct((c, total), x.dtype),
        scratch_shapes=[pltpu.VMEM((c, STEP), x.dtype)],
    )(x)


def kernel(input_time_series):
    return _sc2_kernel(input_time_series)


# speedup vs baseline: 1.0356x; 1.0356x over previous
import functools

import jax
import jax.numpy as jnp
from jax import lax
from jax.experimental import pallas as pl
from jax.experimental.pallas import tpu as pltpu
from jax.experimental.pallas import tpu_sc as plsc

WIDTH = 512
STEP = 256
LANES = 16


SC_G = 8
SC_NBUF = 4
SC_LOOK = 2
SC_OWAIT = 2


SC_OG = 2
SC_OBUF = 2


def _sc_body(x_hbm, out_hbm, *refs, n, ng):
    wins = refs[0:SC_NBUF]
    outs = refs[SC_NBUF:SC_NBUF + SC_OBUF]
    carry_v = refs[SC_NBUF + SC_OBUF]
    isems = refs[SC_NBUF + SC_OBUF + 1:2 * SC_NBUF + SC_OBUF + 1]
    osems = refs[2 * SC_NBUF + SC_OBUF + 1:2 * SC_NBUF + 2 * SC_OBUF + 1]
    ch = lax.axis_index("s") * 2 + lax.axis_index("c")

    def in_cp(g, b):
        return pltpu.make_async_copy(
            x_hbm.at[pl.ds(g * SC_G, SC_G), pl.ds(ch, 1), :], wins[b], isems[b])

    def out_cp(q, ob):
        flush = SC_OG * SC_G * STEP
        return pltpu.make_async_copy(
            outs[ob], out_hbm.at[pl.ds(ch, 1), pl.ds(q * flush, flush)],
            osems[ob])

    zero = jnp.zeros((LANES,), jnp.float32)
    for i in range(STEP // LANES):
        carry_v[0, pl.ds(i * LANES, LANES)] = zero

    for d in range(SC_LOOK):
        in_cp(d, d).start()

    def outer(gq, _):
        for b in range(SC_NBUF):
            g = gq * SC_NBUF + b
            ob = b // SC_OG
            q = gq * (SC_NBUF // SC_OG) + ob

            @pl.when(g + SC_LOOK < ng)
            def _():
                in_cp(g + SC_LOOK, (b + SC_LOOK) % SC_NBUF).start()

            in_cp(g, b).wait()

            if b % SC_OG == 0:
                @pl.when(q >= SC_OBUF)
                def _():
                    out_cp(q - SC_OBUF, ob).wait()

            win_v = wins[b]
            out_v = outs[ob]
            col0 = (b % SC_OG) * SC_G * STEP
            for k in range(SC_G):
                for i in range(STEP // LANES):
                    off = i * LANES
                    a = win_v[k, 0, pl.ds(off, LANES)]
                    if k == 0:
                        b_half = carry_v[0, pl.ds(off, LANES)]
                    else:
                        b_half = win_v[k - 1, 0, pl.ds(STEP + off, LANES)]
                    out_v[0, pl.ds(col0 + k * STEP + off, LANES)] = a + b_half
            for i in range(STEP // LANES):
                off = i * LANES
                carry_v[0, pl.ds(off, LANES)] = (
                    win_v[SC_G - 1, 0, pl.ds(STEP + off, LANES)])
            if b % SC_OG == SC_OG - 1:
                out_cp(q, ob).start()
        return 0

    lax.fori_loop(0, ng // SC_NBUF, outer, 0)
    nq = ng // SC_OG
    for d in range(SC_OBUF):
        q = nq - SC_OBUF + d
        out_cp(q, q % SC_OBUF).wait()
    pltpu.sync_copy(carry_v, out_hbm.at[pl.ds(ch, 1), pl.ds(n * STEP, STEP)])


def _sc_kernel(x):
    n, c, w = x.shape
    total = (n - 1) * STEP + w
    ng = n // SC_G
    mesh = plsc.VectorSubcoreMesh(core_axis_name="c", subcore_axis_name="s")
    kfn = pl.kernel(
        functools.partial(_sc_body, n=n, ng=ng),
        out_type=jax.ShapeDtypeStruct((c, total), x.dtype),
        mesh=mesh,
        scratch_types=(
            [pltpu.VMEM((SC_G, 1, WIDTH), jnp.float32)] * SC_NBUF
            + [pltpu.VMEM((1, SC_OG * SC_G * STEP), jnp.float32)] * SC_OBUF
            + [pltpu.VMEM((1, STEP), jnp.float32)]
            + [pltpu.SemaphoreType.DMA] * SC_NBUF
            + [pltpu.SemaphoreType.DMA] * SC_OBUF
        ),
    )
    return kfn(x)


SC2_G = 4
SC2_NT = 2
SC2_CPW = 2


def _sc2_body(x_hbm, out_hbm, *refs, n, ng, nw):
    wins = refs[0:SC_NBUF]
    outs = refs[SC_NBUF:2 * SC_NBUF]
    carry_v = refs[2 * SC_NBUF]
    tail_v = refs[2 * SC_NBUF + 1]
    isems = refs[2 * SC_NBUF + 2:3 * SC_NBUF + 2]
    osems = refs[3 * SC_NBUF + 2:4 * SC_NBUF + 2]
    cg = lax.axis_index("s")
    t = lax.axis_index("c")
    w0 = t * nw
    c0 = cg * SC2_CPW

    def in_cp(g, b):
        return pltpu.make_async_copy(
            x_hbm.at[pl.ds(w0 + g * SC2_G, SC2_G), pl.ds(c0, SC2_CPW), :],
            wins[b], isems[b])

    def out_cp(g, b):
        return pltpu.make_async_copy(
            outs[b],
            out_hbm.at[pl.ds(c0, SC2_CPW),
                       pl.ds((w0 + g * SC2_G) * STEP, SC2_G * STEP)],
            osems[b])

    zero = jnp.zeros((LANES,), jnp.float32)

    @pl.when(t == 0)
    def _():
        for cl in range(SC2_CPW):
            for i in range(STEP // LANES):
                carry_v[0, cl, pl.ds(i * LANES, LANES)] = zero

    @pl.when(t > 0)
    def _():
        pltpu.sync_copy(
            x_hbm.at[pl.ds(w0 - 1, 1), pl.ds(c0, SC2_CPW), pl.ds(STEP, STEP)],
            carry_v)

    for d in range(SC_LOOK):
        in_cp(d, d).start()

    def outer(gq, _):
        for b in range(SC_NBUF):
            g = gq * SC_NBUF + b

            @pl.when(g + SC_LOOK < ng)
            def _():
                in_cp(g + SC_LOOK, (b + SC_LOOK) % SC_NBUF).start()

            in_cp(g, b).wait()

            @pl.when(g >= SC_OWAIT)
            def _():
                out_cp(g - SC_OWAIT, (b - SC_OWAIT) % SC_NBUF).wait()

            win_v = wins[b]
            out_v = outs[b]
            for k in range(SC2_G):
                for cl in range(SC2_CPW):
                    for i in range(STEP // LANES):
                        off = i * LANES
                        a = win_v[k, cl, pl.ds(off, LANES)]
                        if k == 0:
                            b_half = carry_v[0, cl, pl.ds(off, LANES)]
                        else:
                            b_half = win_v[k - 1, cl, pl.ds(STEP + off, LANES)]
                        out_v[cl, pl.ds(k * STEP + off, LANES)] = a + b_half
            for cl in range(SC2_CPW):
                for i in range(STEP // LANES):
                    off = i * LANES
                    carry_v[0, cl, pl.ds(off, LANES)] = (
                        win_v[SC2_G - 1, cl, pl.ds(STEP + off, LANES)])
            out_cp(g, b).start()
        return 0

    lax.fori_loop(0, ng // SC_NBUF, outer, 0)
    for d in range(SC_OWAIT):
        g = ng - SC_OWAIT + d
        out_cp(g, g % SC_NBUF).wait()

    @pl.when(t == SC2_NT - 1)
    def _():
        for cl in range(SC2_CPW):
            for i in range(STEP // LANES):
                off = i * LANES
                tail_v[cl, pl.ds(off, LANES)] = carry_v[0, cl, pl.ds(off, LANES)]
        pltpu.sync_copy(
            tail_v, out_hbm.at[pl.ds(c0, SC2_CPW), pl.ds(n * STEP, STEP)])


def _sc2_kernel(x):
    n, c, w = x.shape
    total = (n - 1) * STEP + w
    nw = n // SC2_NT
    ng = nw // SC2_G
    mesh = plsc.VectorSubcoreMesh(core_axis_name="c", subcore_axis_name="s")
    kfn = pl.kernel(
        functools.partial(_sc2_body, n=n, ng=ng, nw=nw),
        out_type=jax.ShapeDtypeStruct((c, total), x.dtype),
        mesh=mesh,
        scratch_types=(
            [pltpu.VMEM((SC2_G, SC2_CPW, WIDTH), jnp.float32)] * SC_NBUF
            + [pltpu.VMEM((SC2_CPW, SC2_G * STEP), jnp.float32)] * SC_NBUF
            + [pltpu.VMEM((1, SC2_CPW, STEP), jnp.float32)]
            + [pltpu.VMEM((SC2_CPW, STEP), jnp.float32)]
            + [pltpu.SemaphoreType.DMA] * (2 * SC_NBUF)
        ),
    )
    return kfn(x)


G = 8


def _tc_body(x_ref, o_ref, carry_ref, *, nb):
    j = pl.program_id(0)

    @pl.when(j < nb)
    def _main():
        a0 = x_ref[0, :, :STEP]
        o_ref[:, :STEP] = jnp.where(j == 0, a0, a0 + carry_ref[...])
        for k in range(1, G):
            o_ref[:, k * STEP:(k + 1) * STEP] = (
                x_ref[k, :, :STEP] + x_ref[k - 1, :, STEP:])
        carry_ref[...] = x_ref[G - 1, :, STEP:]

    @pl.when(j == nb)
    def _tail():
        o_ref[:, :STEP] = carry_ref[...]


def _tc_kernel(x):
    n, c, w = x.shape
    total = (n - 1) * STEP + w
    nb = n // G
    return pl.pallas_call(
        functools.partial(_tc_body, nb=nb),
        grid=(nb + 1,),
        in_specs=[pl.BlockSpec((G, c, w), lambda j: (jnp.minimum(j, nb - 1), 0, 0))],
        out_specs=pl.BlockSpec((c, G * STEP), lambda j: (0, j)),
        out_shape=jax.ShapeDtypeStruct((c, total), x.dtype),
        scratch_shapes=[pltpu.VMEM((c, STEP), x.dtype)],
    )(x)


def kernel(input_time_series):
    return _sc_kernel(input_time_series)


# speedup vs baseline: 1.3700x; 1.3229x over previous
import functools

import jax
import jax.numpy as jnp
from jax import lax
from jax.experimental import pallas as pl
from jax.experimental.pallas import tpu as pltpu
from jax.experimental.pallas import tpu_sc as plsc

WIDTH = 512
STEP = 256
LANES = 16

SC_G = 8
SC_NBUF = 4
SC_LOOK = 2
SC_OWAIT = 2


def _sc_body(x_hbm, out_hbm, *refs, n, ng):
    wins = refs[0:SC_NBUF]
    outs = refs[SC_NBUF:2 * SC_NBUF]
    carry_v = refs[2 * SC_NBUF]
    isems = refs[2 * SC_NBUF + 1:3 * SC_NBUF + 1]
    osems = refs[3 * SC_NBUF + 1:4 * SC_NBUF + 1]
    ch = lax.axis_index("s") * 2 + lax.axis_index("c")

    def in_cp(g, b):
        return pltpu.make_async_copy(
            x_hbm.at[pl.ds(g * SC_G, SC_G), pl.ds(ch, 1), :], wins[b], isems[b])

    def out_cp(g, b):
        return pltpu.make_async_copy(
            outs[b], out_hbm.at[pl.ds(ch, 1), pl.ds(g * SC_G * STEP, SC_G * STEP)],
            osems[b])

    zero = jnp.zeros((LANES,), jnp.float32)
    for i in range(STEP // LANES):
        carry_v[0, pl.ds(i * LANES, LANES)] = zero

    for d in range(SC_LOOK):
        in_cp(d, d).start()

    def outer(gq, _):
        for b in range(SC_NBUF):
            g = gq * SC_NBUF + b

            @pl.when(g + SC_LOOK < ng)
            def _():
                in_cp(g + SC_LOOK, (b + SC_LOOK) % SC_NBUF).start()

            in_cp(g, b).wait()

            @pl.when(g >= SC_OWAIT)
            def _():
                out_cp(g - SC_OWAIT, (b - SC_OWAIT) % SC_NBUF).wait()

            win_v = wins[b]
            out_v = outs[b]
            for k in range(SC_G):
                for i in range(STEP // LANES):
                    off = i * LANES
                    a = win_v[k, 0, pl.ds(off, LANES)]
                    if k == 0:
                        b_half = carry_v[0, pl.ds(off, LANES)]
                    else:
                        b_half = win_v[k - 1, 0, pl.ds(STEP + off, LANES)]
                    out_v[0, pl.ds(k * STEP + off, LANES)] = a + b_half
            for i in range(STEP // LANES):
                off = i * LANES
                carry_v[0, pl.ds(off, LANES)] = (
                    win_v[SC_G - 1, 0, pl.ds(STEP + off, LANES)])
            out_cp(g, b).start()
        return 0

    lax.fori_loop(0, ng // SC_NBUF, outer, 0)
    for d in range(SC_OWAIT):
        g = ng - SC_OWAIT + d
        out_cp(g, g % SC_NBUF).wait()
    pltpu.sync_copy(carry_v, out_hbm.at[pl.ds(ch, 1), pl.ds(n * STEP, STEP)])


def kernel(input_time_series):
    x = input_time_series
    n, c, w = x.shape
    total = (n - 1) * STEP + w
    ng = n // SC_G
    mesh = plsc.VectorSubcoreMesh(core_axis_name="c", subcore_axis_name="s")
    kfn = pl.kernel(
        functools.partial(_sc_body, n=n, ng=ng),
        out_type=jax.ShapeDtypeStruct((c, total), x.dtype),
        mesh=mesh,
        scratch_types=(
            [pltpu.VMEM((SC_G, 1, WIDTH), jnp.float32)] * SC_NBUF
            + [pltpu.VMEM((1, SC_G * STEP), jnp.float32)] * SC_NBUF
            + [pltpu.VMEM((1, STEP), jnp.float32)]
            + [pltpu.SemaphoreType.DMA] * (2 * SC_NBUF)
        ),
    )
    return kfn(x)
